# baseline (speedup 1.0000x reference)
import functools

import jax
import jax.numpy as jnp
from jax import lax
from jax.experimental import pallas as pl
from jax.experimental.pallas import tpu as pltpu
from jax.experimental.pallas import tpu_sc as plsc

N_NODES = 10000
N_PAD = 10240
N_EDGES = 320000
IN_F = 128
HID_F = 256
FC_F = 256
N_CLS = 40

NUM_SUBCORES = 16
ROWS_PER_TILE = N_PAD // NUM_SUBCORES
E_PER_TILE = N_EDGES // NUM_SUBCORES
CHUNK = 128
NCH = 160
NBUF = 2
G = 32
E_PAD = NCH * CHUNK * NUM_SUBCORES
HALF_F = 128

_sc_mesh = functools.partial(
    plsc.VectorSubcoreMesh, core_axis_name="c", subcore_axis_name="s")


def _deg_body(ei_flat_hbm, deg2_hbm, idx_v, hist_v, vbuf_v, sh2, _sem):
    c = lax.axis_index("c")
    s = lax.axis_index("s")

    def zh(i, carry):
        hist_v[pl.ds(i * 16, 16)] = jnp.zeros((16,), jnp.float32)
        return carry

    lax.fori_loop(0, N_PAD // 16, zh, 0)

    base = c * N_EDGES + s * E_PER_TILE
    pltpu.sync_copy(ei_flat_hbm.at[pl.ds(base, E_PER_TILE)], idx_v)
    ones = jnp.ones((16,), jnp.float32)

    def step(i, carry):
        iv = idx_v[pl.ds(i * 16, 16)]
        plsc.addupdate_scatter(hist_v, [iv], ones)
        return carry

    lax.fori_loop(0, E_PER_TILE // 16, step, 0)

    pltpu.sync_copy(hist_v, sh2.at[s])
    plsc.subcore_barrier()
    pltpu.sync_copy(sh2.at[:, pl.ds(s * ROWS_PER_TILE, ROWS_PER_TILE)], vbuf_v)

    def red(j, carry):
        acc = jnp.zeros((16,), jnp.float32)
        for t in range(NUM_SUBCORES):
            acc = acc + vbuf_v[t, pl.ds(j * 16, 16)]
        hist_v[pl.ds(j * 16, 16)] = acc
        return carry

    lax.fori_loop(0, ROWS_PER_TILE // 16, red, 0)
    pltpu.sync_copy(hist_v.at[pl.ds(0, ROWS_PER_TILE)],
                    deg2_hbm.at[pl.ds(c * N_PAD + s * ROWS_PER_TILE,
                                      ROWS_PER_TILE)])


def _degrees(ei_flat):
    return pl.kernel(
        _deg_body,
        out_type=jax.ShapeDtypeStruct((2 * N_PAD,), jnp.float32),
        mesh=_sc_mesh(),
        scratch_types=[
            pltpu.VMEM((E_PER_TILE,), jnp.int32),
            pltpu.VMEM((N_PAD,), jnp.float32),
            pltpu.VMEM((NUM_SUBCORES, ROWS_PER_TILE), jnp.float32),
            pltpu.VMEM_SHARED((NUM_SUBCORES, N_PAD), jnp.float32),
            pltpu.SemaphoreType.DMA,
        ],
        compiler_params=pltpu.CompilerParams(needs_layout_passes=False),
    )(ei_flat)


def _agg_body(ha_hbm, hb_hbm, src2_hbm, dst2_hbm, outa_hbm, outb_hbm,
              idxs_v, idxd_v, rows_v, acc_sh, gsems, ssems):
    c = lax.axis_index("c")
    s = lax.axis_index("s")

    def zrow(i, carry):
        for b in range(NBUF):
            for j in range(HALF_F // 16):
                rows_v[b, i, pl.ds(j * 16, 16)] = jnp.zeros((16,), jnp.float32)
        return carry

    lax.fori_loop(0, CHUNK, zrow, 0)

    def zero_slice(i, carry):
        pltpu.sync_copy(rows_v.at[0],
                        acc_sh.at[pl.ds(s * ROWS_PER_TILE + i * CHUNK, CHUNK)])
        return carry

    lax.fori_loop(0, ROWS_PER_TILE // CHUNK, zero_slice, 0)
    plsc.subcore_barrier()

    def run(h_hbm):
        def group_body(g, carry):
            gb = s * NCH + g * G
            pltpu.sync_copy(src2_hbm.at[pl.ds(gb, G)], idxs_v)
            pltpu.sync_copy(dst2_hbm.at[pl.ds(gb, G)], idxd_v)
            for b in range(NBUF):
                pltpu.async_copy(h_hbm.at[idxs_v.at[b]], rows_v.at[b],
                                 gsems.at[b])

            def chunk_grp(i, carry2):
                for b in range(NBUF):
                    k = i * NBUF + b
                    ob = 1 - b
                    pltpu.make_async_copy(
                        h_hbm.at[idxs_v.at[k]], rows_v.at[b],
                        gsems.at[b]).wait()
                    pltpu.async_copy(rows_v.at[b], acc_sh.at[idxd_v.at[k]],
                                     ssems.at[b], add=True)

                    @pl.when(k >= 1)
                    def _():
                        pltpu.make_async_copy(
                            rows_v.at[ob], acc_sh.at[idxd_v.at[k - 1]],
                            ssems.at[ob]).wait()

                    @pl.when(jnp.logical_and(k >= 1, k + 1 < G))
                    def _():
                        pltpu.async_copy(h_hbm.at[idxs_v.at[k + 1]],
                                         rows_v.at[ob], gsems.at[ob])
                return carry2

            lax.fori_loop(0, G // NBUF, chunk_grp, 0)
            pltpu.make_async_copy(rows_v.at[1], acc_sh.at[idxd_v.at[G - 1]],
                                  ssems.at[1]).wait()
            return carry

        lax.fori_loop(0, NCH // G, group_body, 0)

    pl.when(c == 0)(lambda: run(ha_hbm))
    pl.when(c == 1)(lambda: run(hb_hbm))
    plsc.subcore_barrier()

    def out_copy(o_hbm):
        pltpu.sync_copy(acc_sh.at[pl.ds(s * ROWS_PER_TILE, ROWS_PER_TILE)],
                        o_hbm.at[pl.ds(s * ROWS_PER_TILE, ROWS_PER_TILE)])

    pl.when(c == 0)(lambda: out_copy(outa_hbm))
    pl.when(c == 1)(lambda: out_copy(outb_hbm))


def _aggregate(ha, hb, src2, dst2):
    return pl.kernel(
        _agg_body,
        out_type=(
            jax.ShapeDtypeStruct((N_PAD, HALF_F), jnp.float32),
            jax.ShapeDtypeStruct((N_PAD, HALF_F), jnp.float32),
        ),
        mesh=_sc_mesh(),
        scratch_types=[
            pltpu.VMEM((G, CHUNK), jnp.int32),
            pltpu.VMEM((G, CHUNK), jnp.int32),
            pltpu.VMEM((NBUF, CHUNK, HALF_F), jnp.float32),
            pltpu.VMEM_SHARED((N_PAD, HALF_F), jnp.float32),
            pltpu.SemaphoreType.DMA((NBUF,)),
            pltpu.SemaphoreType.DMA((NBUF,)),
        ],
    )(ha, hb, src2, dst2)


BM = 1024
GRID = N_PAD // BM


def _tc1_body(x_ref, dego_ref, w1_ref, ha_ref, hb_ref):
    norm = lax.rsqrt(jnp.maximum(dego_ref[...], 1.0))
    xs = x_ref[...] * norm
    h = jnp.dot(xs, w1_ref[...], preferred_element_type=jnp.float32)
    ha_ref[...] = h[:, :HALF_F]
    hb_ref[...] = h[:, HALF_F:]


def _tc1(x, dego, w1):
    return pl.pallas_call(
        _tc1_body,
        grid=(GRID,),
        in_specs=[
            pl.BlockSpec((BM, IN_F), lambda i: (i, 0)),
            pl.BlockSpec((BM, 1), lambda i: (i, 0)),
            pl.BlockSpec((IN_F, HID_F), lambda i: (0, 0)),
        ],
        out_specs=[
            pl.BlockSpec((BM, HALF_F), lambda i: (i, 0)),
            pl.BlockSpec((BM, HALF_F), lambda i: (i, 0)),
        ],
        out_shape=[
            jax.ShapeDtypeStruct((N_PAD, HALF_F), jnp.float32),
            jax.ShapeDtypeStruct((N_PAD, HALF_F), jnp.float32),
        ],
    )(x, dego, w1)


def _tc2_body(aa_ref, ab_ref, degi_ref, dego_ref, b1_ref, w2_ref,
              ha_ref, hb_ref):
    ni = lax.rsqrt(jnp.maximum(degi_ref[...], 1.0))
    no = lax.rsqrt(jnp.maximum(dego_ref[...], 1.0))
    b = b1_ref[...]
    ta = jax.nn.relu(aa_ref[...] * ni + b[:, :HALF_F]) * no
    tb = jax.nn.relu(ab_ref[...] * ni + b[:, HALF_F:]) * no
    w = w2_ref[...]
    h = (jnp.dot(ta, w[:HALF_F, :], preferred_element_type=jnp.float32)
         + jnp.dot(tb, w[HALF_F:, :], preferred_element_type=jnp.float32))
    ha_ref[...] = h[:, :HALF_F]
    hb_ref[...] = h[:, HALF_F:]


def _tc2(aa, ab, degi, dego, b1, w2):
    return pl.pallas_call(
        _tc2_body,
        grid=(GRID,),
        in_specs=[
            pl.BlockSpec((BM, HALF_F), lambda i: (i, 0)),
            pl.BlockSpec((BM, HALF_F), lambda i: (i, 0)),
            pl.BlockSpec((BM, 1), lambda i: (i, 0)),
            pl.BlockSpec((BM, 1), lambda i: (i, 0)),
            pl.BlockSpec((1, HID_F), lambda i: (0, 0)),
            pl.BlockSpec((HID_F, FC_F), lambda i: (0, 0)),
        ],
        out_specs=[
            pl.BlockSpec((BM, HALF_F), lambda i: (i, 0)),
            pl.BlockSpec((BM, HALF_F), lambda i: (i, 0)),
        ],
        out_shape=[
            jax.ShapeDtypeStruct((N_PAD, HALF_F), jnp.float32),
            jax.ShapeDtypeStruct((N_PAD, HALF_F), jnp.float32),
        ],
    )(aa, ab, degi, dego, b1, w2)


def _tc3_body(aa_ref, ab_ref, degi_ref, b2_ref, wfc_ref, bfc_ref, out_ref):
    ni = lax.rsqrt(jnp.maximum(degi_ref[...], 1.0))
    b = b2_ref[...]
    ta = jax.nn.relu(aa_ref[...] * ni + b[:, :HALF_F])
    tb = jax.nn.relu(ab_ref[...] * ni + b[:, HALF_F:])
    w = wfc_ref[...]
    out_ref[...] = (jnp.dot(ta, w[:HALF_F, :], preferred_element_type=jnp.float32)
                    + jnp.dot(tb, w[HALF_F:, :], preferred_element_type=jnp.float32)
                    + bfc_ref[...])


def _tc3(aa, ab, degi, b2, wfc, bfc):
    return pl.pallas_call(
        _tc3_body,
        grid=(GRID,),
        in_specs=[
            pl.BlockSpec((BM, HALF_F), lambda i: (i, 0)),
            pl.BlockSpec((BM, HALF_F), lambda i: (i, 0)),
            pl.BlockSpec((BM, 1), lambda i: (i, 0)),
            pl.BlockSpec((1, FC_F), lambda i: (0, 0)),
            pl.BlockSpec((FC_F, N_CLS), lambda i: (0, 0)),
            pl.BlockSpec((1, N_CLS), lambda i: (0, 0)),
        ],
        out_specs=pl.BlockSpec((BM, N_CLS), lambda i: (i, 0)),
        out_shape=jax.ShapeDtypeStruct((N_PAD, N_CLS), jnp.float32),
    )(aa, ab, degi, b2, wfc, bfc)


def kernel(x, edge_index, W1, b1, W2, b2, Wfc, bfc):
    ei = edge_index.astype(jnp.int32)
    src = jnp.ravel(ei[0])
    dst = jnp.ravel(ei[1])
    xp = jnp.pad(x, ((0, N_PAD - N_NODES), (0, 0)))

    pad_idx = (jnp.arange(E_PAD - N_EDGES, dtype=jnp.int32)
               % (N_PAD - N_NODES)) + N_NODES
    src2 = jnp.concatenate([src, pad_idx]).reshape(E_PAD // CHUNK, CHUNK)
    dst2 = jnp.concatenate([dst, pad_idx]).reshape(E_PAD // CHUNK, CHUNK)

    deg2 = _degrees(jnp.concatenate([src, dst]))
    dego = deg2[:N_PAD].reshape(N_PAD, 1)
    degi = deg2[N_PAD:].reshape(N_PAD, 1)

    ha, hb = _tc1(xp, dego, W1)
    aa, ab = _aggregate(ha, hb, src2, dst2)
    ha2, hb2 = _tc2(aa, ab, degi, dego, b1.reshape(1, HID_F), W2)
    aa2, ab2 = _aggregate(ha2, hb2, src2, dst2)
    out = _tc3(aa2, ab2, degi, b2.reshape(1, FC_F), Wfc, bfc.reshape(1, N_CLS))
    return out[:N_NODES]


# speedup vs baseline: 1.0309x; 1.0309x over previous
import functools

import jax
import jax.numpy as jnp
from jax import lax
from jax.experimental import pallas as pl
from jax.experimental.pallas import tpu as pltpu
from jax.experimental.pallas import tpu_sc as plsc

N_NODES = 10000
N_PAD = 10240
N_EDGES = 320000
IN_F = 128
HID_F = 256
FC_F = 256
N_CLS = 40

NUM_SUBCORES = 16
ROWS_PER_TILE = N_PAD // NUM_SUBCORES
E_PER_TILE = N_EDGES // NUM_SUBCORES
CHUNK = 64
NCH = 320
NBUF = 4
G = 32
E_PAD = NCH * CHUNK * NUM_SUBCORES
HALF_F = 128

_sc_mesh = functools.partial(
    plsc.VectorSubcoreMesh, core_axis_name="c", subcore_axis_name="s")


def _deg_body(ei_flat_hbm, deg2_hbm, idx_v, hist_v, vbuf_v, sh2, _sem):
    c = lax.axis_index("c")
    s = lax.axis_index("s")

    def zh(i, carry):
        hist_v[pl.ds(i * 16, 16)] = jnp.zeros((16,), jnp.float32)
        return carry

    lax.fori_loop(0, N_PAD // 16, zh, 0)

    base = c * N_EDGES + s * E_PER_TILE
    pltpu.sync_copy(ei_flat_hbm.at[pl.ds(base, E_PER_TILE)], idx_v)
    ones = jnp.ones((16,), jnp.float32)

    def step(i, carry):
        iv = idx_v[pl.ds(i * 16, 16)]
        plsc.addupdate_scatter(hist_v, [iv], ones)
        return carry

    lax.fori_loop(0, E_PER_TILE // 16, step, 0)

    pltpu.sync_copy(hist_v, sh2.at[s])
    plsc.subcore_barrier()
    pltpu.sync_copy(sh2.at[:, pl.ds(s * ROWS_PER_TILE, ROWS_PER_TILE)], vbuf_v)

    def red(j, carry):
        acc = jnp.zeros((16,), jnp.float32)
        for t in range(NUM_SUBCORES):
            acc = acc + vbuf_v[t, pl.ds(j * 16, 16)]
        hist_v[pl.ds(j * 16, 16)] = acc
        return carry

    lax.fori_loop(0, ROWS_PER_TILE // 16, red, 0)
    pltpu.sync_copy(hist_v.at[pl.ds(0, ROWS_PER_TILE)],
                    deg2_hbm.at[pl.ds(c * N_PAD + s * ROWS_PER_TILE,
                                      ROWS_PER_TILE)])


def _degrees(ei_flat):
    return pl.kernel(
        _deg_body,
        out_type=jax.ShapeDtypeStruct((2 * N_PAD,), jnp.float32),
        mesh=_sc_mesh(),
        scratch_types=[
            pltpu.VMEM((E_PER_TILE,), jnp.int32),
            pltpu.VMEM((N_PAD,), jnp.float32),
            pltpu.VMEM((NUM_SUBCORES, ROWS_PER_TILE), jnp.float32),
            pltpu.VMEM_SHARED((NUM_SUBCORES, N_PAD), jnp.float32),
            pltpu.SemaphoreType.DMA,
        ],
        compiler_params=pltpu.CompilerParams(needs_layout_passes=False),
    )(ei_flat)


def _agg_body(ha_hbm, hb_hbm, src2_hbm, dst2_hbm, outa_hbm, outb_hbm,
              idxs_v, idxd_v, rows_v, acc_sh, gsems, ssems):
    c = lax.axis_index("c")
    s = lax.axis_index("s")

    def zrow(i, carry):
        for b in range(NBUF):
            for j in range(HALF_F // 16):
                rows_v[b, i, pl.ds(j * 16, 16)] = jnp.zeros((16,), jnp.float32)
        return carry

    lax.fori_loop(0, CHUNK, zrow, 0)

    def zero_slice(i, carry):
        pltpu.sync_copy(rows_v.at[0],
                        acc_sh.at[pl.ds(s * ROWS_PER_TILE + i * CHUNK, CHUNK)])
        return carry

    lax.fori_loop(0, ROWS_PER_TILE // CHUNK, zero_slice, 0)
    plsc.subcore_barrier()

    def run(h_hbm):
        def group_body(g, carry):
            gb = s * NCH + g * G
            pltpu.sync_copy(src2_hbm.at[pl.ds(gb, G)], idxs_v)
            pltpu.sync_copy(dst2_hbm.at[pl.ds(gb, G)], idxd_v)
            for b in range(2):
                pltpu.async_copy(h_hbm.at[idxs_v.at[b]], rows_v.at[b],
                                 gsems.at[b])

            def chunk_grp(i, carry2):
                for b in range(NBUF):
                    j = i * NBUF + b
                    rb = (b + 2) % NBUF
                    pltpu.make_async_copy(
                        h_hbm.at[idxs_v.at[j]], rows_v.at[b],
                        gsems.at[b]).wait()
                    pltpu.async_copy(rows_v.at[b], acc_sh.at[idxd_v.at[j]],
                                     ssems.at[b], add=True)

                    @pl.when(j >= 2)
                    def _():
                        pltpu.make_async_copy(
                            rows_v.at[rb], acc_sh.at[idxd_v.at[j - 2]],
                            ssems.at[rb]).wait()

                    @pl.when(j + 2 < G)
                    def _():
                        pltpu.async_copy(h_hbm.at[idxs_v.at[j + 2]],
                                         rows_v.at[rb], gsems.at[rb])
                return carry2

            lax.fori_loop(0, G // NBUF, chunk_grp, 0)
            pltpu.make_async_copy(rows_v.at[(G - 2) % NBUF],
                                  acc_sh.at[idxd_v.at[G - 2]],
                                  ssems.at[(G - 2) % NBUF]).wait()
            pltpu.make_async_copy(rows_v.at[(G - 1) % NBUF],
                                  acc_sh.at[idxd_v.at[G - 1]],
                                  ssems.at[(G - 1) % NBUF]).wait()
            return carry

        lax.fori_loop(0, NCH // G, group_body, 0)

    pl.when(c == 0)(lambda: run(ha_hbm))
    pl.when(c == 1)(lambda: run(hb_hbm))
    plsc.subcore_barrier()

    def out_copy(o_hbm):
        pltpu.sync_copy(acc_sh.at[pl.ds(s * ROWS_PER_TILE, ROWS_PER_TILE)],
                        o_hbm.at[pl.ds(s * ROWS_PER_TILE, ROWS_PER_TILE)])

    pl.when(c == 0)(lambda: out_copy(outa_hbm))
    pl.when(c == 1)(lambda: out_copy(outb_hbm))


def _aggregate(ha, hb, src2, dst2):
    return pl.kernel(
        _agg_body,
        out_type=(
            jax.ShapeDtypeStruct((N_PAD, HALF_F), jnp.float32),
            jax.ShapeDtypeStruct((N_PAD, HALF_F), jnp.float32),
        ),
        mesh=_sc_mesh(),
        scratch_types=[
            pltpu.VMEM((G, CHUNK), jnp.int32),
            pltpu.VMEM((G, CHUNK), jnp.int32),
            pltpu.VMEM((NBUF, CHUNK, HALF_F), jnp.float32),
            pltpu.VMEM_SHARED((N_PAD, HALF_F), jnp.float32),
            pltpu.SemaphoreType.DMA((NBUF,)),
            pltpu.SemaphoreType.DMA((NBUF,)),
        ],
    )(ha, hb, src2, dst2)


BM = 1024
GRID = N_PAD // BM


def _tc1_body(x_ref, dego_ref, w1_ref, ha_ref, hb_ref):
    norm = lax.rsqrt(jnp.maximum(dego_ref[...], 1.0))
    xs = x_ref[...] * norm
    h = jnp.dot(xs, w1_ref[...], preferred_element_type=jnp.float32)
    ha_ref[...] = h[:, :HALF_F]
    hb_ref[...] = h[:, HALF_F:]


def _tc1(x, dego, w1):
    return pl.pallas_call(
        _tc1_body,
        grid=(GRID,),
        in_specs=[
            pl.BlockSpec((BM, IN_F), lambda i: (i, 0)),
            pl.BlockSpec((BM, 1), lambda i: (i, 0)),
            pl.BlockSpec((IN_F, HID_F), lambda i: (0, 0)),
        ],
        out_specs=[
            pl.BlockSpec((BM, HALF_F), lambda i: (i, 0)),
            pl.BlockSpec((BM, HALF_F), lambda i: (i, 0)),
        ],
        out_shape=[
            jax.ShapeDtypeStruct((N_PAD, HALF_F), jnp.float32),
            jax.ShapeDtypeStruct((N_PAD, HALF_F), jnp.float32),
        ],
    )(x, dego, w1)


def _tc2_body(aa_ref, ab_ref, degi_ref, dego_ref, b1_ref, w2_ref,
              ha_ref, hb_ref):
    ni = lax.rsqrt(jnp.maximum(degi_ref[...], 1.0))
    no = lax.rsqrt(jnp.maximum(dego_ref[...], 1.0))
    b = b1_ref[...]
    ta = jax.nn.relu(aa_ref[...] * ni + b[:, :HALF_F]) * no
    tb = jax.nn.relu(ab_ref[...] * ni + b[:, HALF_F:]) * no
    w = w2_ref[...]
    h = (jnp.dot(ta, w[:HALF_F, :], preferred_element_type=jnp.float32)
         + jnp.dot(tb, w[HALF_F:, :], preferred_element_type=jnp.float32))
    ha_ref[...] = h[:, :HALF_F]
    hb_ref[...] = h[:, HALF_F:]


def _tc2(aa, ab, degi, dego, b1, w2):
    return pl.pallas_call(
        _tc2_body,
        grid=(GRID,),
        in_specs=[
            pl.BlockSpec((BM, HALF_F), lambda i: (i, 0)),
            pl.BlockSpec((BM, HALF_F), lambda i: (i, 0)),
            pl.BlockSpec((BM, 1), lambda i: (i, 0)),
            pl.BlockSpec((BM, 1), lambda i: (i, 0)),
            pl.BlockSpec((1, HID_F), lambda i: (0, 0)),
            pl.BlockSpec((HID_F, FC_F), lambda i: (0, 0)),
        ],
        out_specs=[
            pl.BlockSpec((BM, HALF_F), lambda i: (i, 0)),
            pl.BlockSpec((BM, HALF_F), lambda i: (i, 0)),
        ],
        out_shape=[
            jax.ShapeDtypeStruct((N_PAD, HALF_F), jnp.float32),
            jax.ShapeDtypeStruct((N_PAD, HALF_F), jnp.float32),
        ],
    )(aa, ab, degi, dego, b1, w2)


def _tc3_body(aa_ref, ab_ref, degi_ref, b2_ref, wfc_ref, bfc_ref, out_ref):
    ni = lax.rsqrt(jnp.maximum(degi_ref[...], 1.0))
    b = b2_ref[...]
    ta = jax.nn.relu(aa_ref[...] * ni + b[:, :HALF_F])
    tb = jax.nn.relu(ab_ref[...] * ni + b[:, HALF_F:])
    w = wfc_ref[...]
    out_ref[...] = (jnp.dot(ta, w[:HALF_F, :], preferred_element_type=jnp.float32)
                    + jnp.dot(tb, w[HALF_F:, :], preferred_element_type=jnp.float32)
                    + bfc_ref[...])


def _tc3(aa, ab, degi, b2, wfc, bfc):
    return pl.pallas_call(
        _tc3_body,
        grid=(GRID,),
        in_specs=[
            pl.BlockSpec((BM, HALF_F), lambda i: (i, 0)),
            pl.BlockSpec((BM, HALF_F), lambda i: (i, 0)),
            pl.BlockSpec((BM, 1), lambda i: (i, 0)),
            pl.BlockSpec((1, FC_F), lambda i: (0, 0)),
            pl.BlockSpec((FC_F, N_CLS), lambda i: (0, 0)),
            pl.BlockSpec((1, N_CLS), lambda i: (0, 0)),
        ],
        out_specs=pl.BlockSpec((BM, N_CLS), lambda i: (i, 0)),
        out_shape=jax.ShapeDtypeStruct((N_PAD, N_CLS), jnp.float32),
    )(aa, ab, degi, b2, wfc, bfc)


def kernel(x, edge_index, W1, b1, W2, b2, Wfc, bfc):
    ei = edge_index.astype(jnp.int32)
    src = jnp.ravel(ei[0])
    dst = jnp.ravel(ei[1])
    xp = jnp.pad(x, ((0, N_PAD - N_NODES), (0, 0)))

    pad_idx = (jnp.arange(E_PAD - N_EDGES, dtype=jnp.int32)
               % (N_PAD - N_NODES)) + N_NODES
    src2 = jnp.concatenate([src, pad_idx]).reshape(E_PAD // CHUNK, CHUNK)
    dst2 = jnp.concatenate([dst, pad_idx]).reshape(E_PAD // CHUNK, CHUNK)

    deg2 = _degrees(jnp.concatenate([src, dst]))
    dego = deg2[:N_PAD].reshape(N_PAD, 1)
    degi = deg2[N_PAD:].reshape(N_PAD, 1)

    ha, hb = _tc1(xp, dego, W1)
    aa, ab = _aggregate(ha, hb, src2, dst2)
    ha2, hb2 = _tc2(aa, ab, degi, dego, b1.reshape(1, HID_F), W2)
    aa2, ab2 = _aggregate(ha2, hb2, src2, dst2)
    out = _tc3(aa2, ab2, degi, b2.reshape(1, FC_F), Wfc, bfc.reshape(1, N_CLS))
    return out[:N_NODES]


# speedup vs baseline: 1.2135x; 1.1771x over previous
import functools

import jax
import jax.numpy as jnp
from jax import lax
from jax.experimental import pallas as pl
from jax.experimental.pallas import tpu as pltpu
from jax.experimental.pallas import tpu_sc as plsc

N_NODES = 10000
N_PAD = 10240
N_EDGES = 320000
IN_F = 128
HID_F = 256
FC_F = 256
N_CLS = 40

NUM_SUBCORES = 16
ROWS_PER_TILE = N_PAD // NUM_SUBCORES
E_PER_TILE = N_EDGES // NUM_SUBCORES
CHUNK = 128
NCH = 160
NBUF = 2
G = 16
NG = NCH // G
E_PAD = NCH * CHUNK * NUM_SUBCORES
HALF_F = 128

_sc_mesh = functools.partial(
    plsc.VectorSubcoreMesh, core_axis_name="c", subcore_axis_name="s")


def _deg_body(ei_flat_hbm, deg2_hbm, idx_v, hist_v, vbuf_v, sh2, _sem):
    c = lax.axis_index("c")
    s = lax.axis_index("s")

    def zh(i, carry):
        hist_v[pl.ds(i * 16, 16)] = jnp.zeros((16,), jnp.float32)
        return carry

    lax.fori_loop(0, N_PAD // 16, zh, 0)

    base = c * N_EDGES + s * E_PER_TILE
    pltpu.sync_copy(ei_flat_hbm.at[pl.ds(base, E_PER_TILE)], idx_v)
    ones = jnp.ones((16,), jnp.float32)

    def step(i, carry):
        iv = idx_v[pl.ds(i * 16, 16)]
        plsc.addupdate_scatter(hist_v, [iv], ones)
        return carry

    lax.fori_loop(0, E_PER_TILE // 16, step, 0)

    pltpu.sync_copy(hist_v, sh2.at[s])
    plsc.subcore_barrier()
    pltpu.sync_copy(sh2.at[:, pl.ds(s * ROWS_PER_TILE, ROWS_PER_TILE)], vbuf_v)

    def red(j, carry):
        acc = jnp.zeros((16,), jnp.float32)
        for t in range(NUM_SUBCORES):
            acc = acc + vbuf_v[t, pl.ds(j * 16, 16)]
        hist_v[pl.ds(j * 16, 16)] = acc
        return carry

    lax.fori_loop(0, ROWS_PER_TILE // 16, red, 0)
    pltpu.sync_copy(hist_v.at[pl.ds(0, ROWS_PER_TILE)],
                    deg2_hbm.at[pl.ds(c * N_PAD + s * ROWS_PER_TILE,
                                      ROWS_PER_TILE)])


def _degrees(ei_flat):
    return pl.kernel(
        _deg_body,
        out_type=jax.ShapeDtypeStruct((2 * N_PAD,), jnp.float32),
        mesh=_sc_mesh(),
        scratch_types=[
            pltpu.VMEM((E_PER_TILE,), jnp.int32),
            pltpu.VMEM((N_PAD,), jnp.float32),
            pltpu.VMEM((NUM_SUBCORES, ROWS_PER_TILE), jnp.float32),
            pltpu.VMEM_SHARED((NUM_SUBCORES, N_PAD), jnp.float32),
            pltpu.SemaphoreType.DMA,
        ],
        compiler_params=pltpu.CompilerParams(needs_layout_passes=False),
    )(ei_flat)


def _agg_body(ha_hbm, hb_hbm, src2_hbm, dst2_hbm, outa_hbm, outb_hbm,
              idxs_v, idxd_v, rows_v, acc_sh, gsems, isems):
    c = lax.axis_index("c")
    s = lax.axis_index("s")

    def zrow(i, carry):
        for b in range(NBUF):
            for j in range(HALF_F // 16):
                rows_v[b, i, pl.ds(j * 16, 16)] = jnp.zeros((16,), jnp.float32)
        return carry

    lax.fori_loop(0, CHUNK, zrow, 0)

    def zero_slice(i, carry):
        pltpu.sync_copy(rows_v.at[0],
                        acc_sh.at[pl.ds(s * ROWS_PER_TILE + i * CHUNK, CHUNK)])
        return carry

    lax.fori_loop(0, ROWS_PER_TILE // CHUNK, zero_slice, 0)
    plsc.subcore_barrier()

    def run(h_hbm):
        tb = s * NCH
        pltpu.sync_copy(src2_hbm.at[pl.ds(tb, G)], idxs_v.at[0])
        pltpu.sync_copy(dst2_hbm.at[pl.ds(tb, G)], idxd_v.at[0])
        pltpu.async_copy(src2_hbm.at[pl.ds(tb + G, G)], idxs_v.at[1],
                         isems.at[1])
        pltpu.async_copy(dst2_hbm.at[pl.ds(tb + G, G)], idxd_v.at[1],
                         isems.at[1])
        for b in range(NBUF):
            pltpu.async_copy(h_hbm.at[idxs_v.at[0, b]], rows_v.at[b],
                             gsems.at[b])

        def one_group(g, p):
            op = 1 - p

            def inner(i, carry):
                for b in range(NBUF):
                    k = i * NBUF + b
                    pltpu.make_async_copy(
                        h_hbm.at[idxs_v.at[p, k]], rows_v.at[b],
                        gsems.at[b]).wait()
                    pltpu.sync_copy(rows_v.at[b], acc_sh.at[idxd_v.at[p, k]],
                                    add=True)
                    pltpu.async_copy(h_hbm.at[idxs_v.at[p, k + 2]],
                                     rows_v.at[b], gsems.at[b])
                return carry

            lax.fori_loop(0, (G - 2) // NBUF, inner, 0)

            pltpu.make_async_copy(h_hbm.at[idxs_v.at[p, G - 2]], rows_v.at[0],
                                  gsems.at[0]).wait()
            pltpu.sync_copy(rows_v.at[0], acc_sh.at[idxd_v.at[p, G - 2]],
                            add=True)

            @pl.when(g + 1 < NG)
            def _():
                pltpu.make_async_copy(src2_hbm.at[pl.ds(tb, G)],
                                      idxs_v.at[op], isems.at[op]).wait()
                pltpu.make_async_copy(dst2_hbm.at[pl.ds(tb, G)],
                                      idxd_v.at[op], isems.at[op]).wait()
                pltpu.async_copy(h_hbm.at[idxs_v.at[op, 0]], rows_v.at[0],
                                 gsems.at[0])

            pltpu.make_async_copy(h_hbm.at[idxs_v.at[p, G - 1]], rows_v.at[1],
                                  gsems.at[1]).wait()
            pltpu.sync_copy(rows_v.at[1], acc_sh.at[idxd_v.at[p, G - 1]],
                            add=True)

            @pl.when(g + 1 < NG)
            def _():
                pltpu.async_copy(h_hbm.at[idxs_v.at[op, 1]], rows_v.at[1],
                                 gsems.at[1])

            @pl.when(g + 2 < NG)
            def _():
                base2 = tb + (g + 2) * G
                pltpu.async_copy(src2_hbm.at[pl.ds(base2, G)], idxs_v.at[p],
                                 isems.at[p])
                pltpu.async_copy(dst2_hbm.at[pl.ds(base2, G)], idxd_v.at[p],
                                 isems.at[p])

        def pair(i, carry):
            one_group(2 * i, 0)
            one_group(2 * i + 1, 1)
            return carry

        lax.fori_loop(0, NG // 2, pair, 0)

    pl.when(c == 0)(lambda: run(ha_hbm))
    pl.when(c == 1)(lambda: run(hb_hbm))
    plsc.subcore_barrier()

    def out_copy(o_hbm):
        pltpu.sync_copy(acc_sh.at[pl.ds(s * ROWS_PER_TILE, ROWS_PER_TILE)],
                        o_hbm.at[pl.ds(s * ROWS_PER_TILE, ROWS_PER_TILE)])

    pl.when(c == 0)(lambda: out_copy(outa_hbm))
    pl.when(c == 1)(lambda: out_copy(outb_hbm))


def _aggregate(ha, hb, src2, dst2):
    return pl.kernel(
        _agg_body,
        out_type=(
            jax.ShapeDtypeStruct((N_PAD, HALF_F), jnp.float32),
            jax.ShapeDtypeStruct((N_PAD, HALF_F), jnp.float32),
        ),
        mesh=_sc_mesh(),
        scratch_types=[
            pltpu.VMEM((2, G, CHUNK), jnp.int32),
            pltpu.VMEM((2, G, CHUNK), jnp.int32),
            pltpu.VMEM((NBUF, CHUNK, HALF_F), jnp.float32),
            pltpu.VMEM_SHARED((N_PAD, HALF_F), jnp.float32),
            pltpu.SemaphoreType.DMA((NBUF,)),
            pltpu.SemaphoreType.DMA((2,)),
        ],
    )(ha, hb, src2, dst2)


BM = 1024
GRID = N_PAD // BM


def _tc1_body(x_ref, dego_ref, w1_ref, ha_ref, hb_ref):
    norm = lax.rsqrt(jnp.maximum(dego_ref[...], 1.0))
    xs = x_ref[...] * norm
    h = jnp.dot(xs, w1_ref[...], preferred_element_type=jnp.float32)
    ha_ref[...] = h[:, :HALF_F]
    hb_ref[...] = h[:, HALF_F:]


def _tc1(x, dego, w1):
    return pl.pallas_call(
        _tc1_body,
        grid=(GRID,),
        in_specs=[
            pl.BlockSpec((BM, IN_F), lambda i: (i, 0)),
            pl.BlockSpec((BM, 1), lambda i: (i, 0)),
            pl.BlockSpec((IN_F, HID_F), lambda i: (0, 0)),
        ],
        out_specs=[
            pl.BlockSpec((BM, HALF_F), lambda i: (i, 0)),
            pl.BlockSpec((BM, HALF_F), lambda i: (i, 0)),
        ],
        out_shape=[
            jax.ShapeDtypeStruct((N_PAD, HALF_F), jnp.float32),
            jax.ShapeDtypeStruct((N_PAD, HALF_F), jnp.float32),
        ],
    )(x, dego, w1)


def _tc2_body(aa_ref, ab_ref, degi_ref, dego_ref, b1_ref, w2_ref,
              ha_ref, hb_ref):
    ni = lax.rsqrt(jnp.maximum(degi_ref[...], 1.0))
    no = lax.rsqrt(jnp.maximum(dego_ref[...], 1.0))
    b = b1_ref[...]
    ta = jax.nn.relu(aa_ref[...] * ni + b[:, :HALF_F]) * no
    tb = jax.nn.relu(ab_ref[...] * ni + b[:, HALF_F:]) * no
    w = w2_ref[...]
    h = (jnp.dot(ta, w[:HALF_F, :], preferred_element_type=jnp.float32)
         + jnp.dot(tb, w[HALF_F:, :], preferred_element_type=jnp.float32))
    ha_ref[...] = h[:, :HALF_F]
    hb_ref[...] = h[:, HALF_F:]


def _tc2(aa, ab, degi, dego, b1, w2):
    return pl.pallas_call(
        _tc2_body,
        grid=(GRID,),
        in_specs=[
            pl.BlockSpec((BM, HALF_F), lambda i: (i, 0)),
            pl.BlockSpec((BM, HALF_F), lambda i: (i, 0)),
            pl.BlockSpec((BM, 1), lambda i: (i, 0)),
            pl.BlockSpec((BM, 1), lambda i: (i, 0)),
            pl.BlockSpec((1, HID_F), lambda i: (0, 0)),
            pl.BlockSpec((HID_F, FC_F), lambda i: (0, 0)),
        ],
        out_specs=[
            pl.BlockSpec((BM, HALF_F), lambda i: (i, 0)),
            pl.BlockSpec((BM, HALF_F), lambda i: (i, 0)),
        ],
        out_shape=[
            jax.ShapeDtypeStruct((N_PAD, HALF_F), jnp.float32),
            jax.ShapeDtypeStruct((N_PAD, HALF_F), jnp.float32),
        ],
    )(aa, ab, degi, dego, b1, w2)


def _tc3_body(aa_ref, ab_ref, degi_ref, b2_ref, wfc_ref, bfc_ref, out_ref):
    ni = lax.rsqrt(jnp.maximum(degi_ref[...], 1.0))
    b = b2_ref[...]
    ta = jax.nn.relu(aa_ref[...] * ni + b[:, :HALF_F])
    tb = jax.nn.relu(ab_ref[...] * ni + b[:, HALF_F:])
    w = wfc_ref[...]
    out_ref[...] = (jnp.dot(ta, w[:HALF_F, :], preferred_element_type=jnp.float32)
                    + jnp.dot(tb, w[HALF_F:, :], preferred_element_type=jnp.float32)
                    + bfc_ref[...])


def _tc3(aa, ab, degi, b2, wfc, bfc):
    return pl.pallas_call(
        _tc3_body,
        grid=(GRID,),
        in_specs=[
            pl.BlockSpec((BM, HALF_F), lambda i: (i, 0)),
            pl.BlockSpec((BM, HALF_F), lambda i: (i, 0)),
            pl.BlockSpec((BM, 1), lambda i: (i, 0)),
            pl.BlockSpec((1, FC_F), lambda i: (0, 0)),
            pl.BlockSpec((FC_F, N_CLS), lambda i: (0, 0)),
            pl.BlockSpec((1, N_CLS), lambda i: (0, 0)),
        ],
        out_specs=pl.BlockSpec((BM, N_CLS), lambda i: (i, 0)),
        out_shape=jax.ShapeDtypeStruct((N_PAD, N_CLS), jnp.float32),
    )(aa, ab, degi, b2, wfc, bfc)


def kernel(x, edge_index, W1, b1, W2, b2, Wfc, bfc):
    ei = edge_index.astype(jnp.int32)
    src = jnp.ravel(ei[0])
    dst = jnp.ravel(ei[1])
    xp = jnp.pad(x, ((0, N_PAD - N_NODES), (0, 0)))

    pad_idx = (jnp.arange(E_PAD - N_EDGES, dtype=jnp.int32)
               % (N_PAD - N_NODES)) + N_NODES
    src2 = jnp.concatenate([src, pad_idx]).reshape(E_PAD // CHUNK, CHUNK)
    dst2 = jnp.concatenate([dst, pad_idx]).reshape(E_PAD // CHUNK, CHUNK)

    deg2 = _degrees(jnp.concatenate([src, dst]))
    dego = deg2[:N_PAD].reshape(N_PAD, 1)
    degi = deg2[N_PAD:].reshape(N_PAD, 1)

    ha, hb = _tc1(xp, dego, W1)
    aa, ab = _aggregate(ha, hb, src2, dst2)
    ha2, hb2 = _tc2(aa, ab, degi, dego, b1.reshape(1, HID_F), W2)
    aa2, ab2 = _aggregate(ha2, hb2, src2, dst2)
    out = _tc3(aa2, ab2, degi, b2.reshape(1, FC_F), Wfc, bfc.reshape(1, N_CLS))
    return out[:N_NODES]
